# Initial kernel scaffold; baseline (speedup 1.0000x reference)
import jax
import jax.numpy as jnp
from jax.experimental import pallas as pl


def kernel(x, hyper_edge_index, W1, b1, W2, b2, W3, b3, Wa, ba):
    raise NotImplementedError("write your pallas kernel here")


# speedup vs baseline: 8.3671x; 8.3671x over previous
import functools

import jax
import jax.numpy as jnp
from jax import lax
from jax.experimental import pallas as pl
from jax.experimental.pallas import tpu as pltpu
from jax.experimental.pallas import tpu_sc as plsc

N = 10000
E = 10000
NNZ = 320000
F = 128
NC = 2
NS = 16
W = 128
NP = 10240
RPT = NP // NS

_MESH = plsc.VectorSubcoreMesh(
    core_axis_name="c", subcore_axis_name="s", num_cores=NC, num_subcores=NS)


def _zero_vmem(buf, rows, cols):
    zero = jnp.zeros((16,), jnp.float32)

    def _row(r, _):
        def _col(j, _):
            buf[r, pl.ds(j * 16, 16)] = zero
            return 0
        return lax.fori_loop(0, cols // 16, _col, 0)

    lax.fori_loop(0, rows, _row, 0)


def _counts_body(idx_hbm, out_hbm, sidx, stidx, ones_v, tones, zb, acc, sem):
    c = lax.axis_index("c")
    s = lax.axis_index("s")
    del sem

    def _zo(j, _):
        zb[pl.ds(j * 16, 16)] = jnp.zeros((16,), jnp.float32)
        return 0
    lax.fori_loop(0, RPT // 16, _zo, 0)
    pltpu.sync_copy(zb, acc.at[pl.ds(s * RPT, RPT)])

    def _one(j, _):
        ones_v[pl.ds(j * 16, 16)] = jnp.ones((16,), jnp.float32)
        return 0
    lax.fori_loop(0, W // 16, _one, 0)

    def _tone(j, _):
        tones[pl.ds(j * 16, 16)] = jnp.ones((16,), jnp.float32)
        return 0
    lax.fori_loop(0, 2, _tone, 0)

    plsc.subcore_barrier()

    npair = NNZ // NS
    base = c * NNZ + s * npair
    nw = npair // W
    tail = npair - nw * W

    def _win(g, _):
        off = base + g * W
        pltpu.sync_copy(idx_hbm.at[pl.ds(off, W)], sidx.at[0])
        pltpu.sync_copy(ones_v, acc.at[sidx.at[0]], add=True)
        return 0
    lax.fori_loop(0, nw, _win, 0)

    off = base + nw * W
    pltpu.sync_copy(idx_hbm.at[pl.ds(off, tail)], stidx.at[0])
    pltpu.sync_copy(tones, acc.at[stidx.at[0]], add=True)

    plsc.subcore_barrier()
    pltpu.sync_copy(acc.at[pl.ds(s * RPT, RPT)],
                    out_hbm.at[c, pl.ds(s * RPT, RPT)])


_counts = pl.kernel(
    _counts_body,
    out_type=jax.ShapeDtypeStruct((NC, NP), jnp.float32),
    mesh=_MESH,
    scratch_types=[
        pltpu.VMEM((2, W), jnp.int32),
        pltpu.VMEM((1, NNZ // NS - (NNZ // NS // W) * W), jnp.int32),
        pltpu.VMEM((W,), jnp.float32),
        pltpu.VMEM((NNZ // NS - (NNZ // NS // W) * W,), jnp.float32),
        pltpu.VMEM((RPT,), jnp.float32),
        pltpu.VMEM_SHARED((NP,), jnp.float32),
        pltpu.SemaphoreType.DMA,
    ],
)


def _prop_body(src_row, dst_row, feat_split, v_hbm, idx_hbm, out_hbm,
               sidx, stidx, rows, trows, zbuf, acc, sem):
    c = lax.axis_index("c")
    s = lax.axis_index("s")

    _zero_vmem(zbuf, RPT // 5, F)
    for k in range(5):
        pltpu.sync_copy(zbuf, acc.at[pl.ds(s * RPT + k * (RPT // 5), RPT // 5)])
    plsc.subcore_barrier()

    if feat_split:
        npair = NNZ // NS
        base = s * npair
        vsrc = v_hbm.at[c]
    else:
        npair = NNZ // (NC * NS)
        base = c * (NNZ // NC) + s * npair
        vsrc = v_hbm
    nw = npair // W
    tail = npair - nw * W

    soff = src_row * NNZ
    doff = dst_row * NNZ

    def _win(g, _):
        off = base + g * W
        pltpu.sync_copy(idx_hbm.at[pl.ds(soff + off, W)], sidx.at[0])
        pltpu.sync_copy(idx_hbm.at[pl.ds(doff + off, W)], sidx.at[1])
        pltpu.async_copy(vsrc.at[sidx.at[0]], rows, sem).wait()
        pltpu.sync_copy(rows, acc.at[sidx.at[1]], add=True)
        return 0
    lax.fori_loop(0, nw, _win, 0)

    if tail:
        off = base + nw * W
        pltpu.sync_copy(idx_hbm.at[pl.ds(soff + off, tail)], stidx.at[0])
        pltpu.sync_copy(idx_hbm.at[pl.ds(doff + off, tail)], stidx.at[1])
        pltpu.async_copy(vsrc.at[stidx.at[0]], trows, sem).wait()
        pltpu.sync_copy(trows, acc.at[stidx.at[1]], add=True)

    plsc.subcore_barrier()
    pltpu.sync_copy(acc.at[pl.ds(s * RPT, RPT)],
                    out_hbm.at[c, pl.ds(s * RPT, RPT)])


@functools.cache
def _make_prop(src_row, dst_row, feat_split):
    npair = NNZ // NS if feat_split else NNZ // (NC * NS)
    tail = npair - (npair // W) * W
    return pl.kernel(
        functools.partial(_prop_body, src_row, dst_row, feat_split),
        out_type=jax.ShapeDtypeStruct((NC, NP, F), jnp.float32),
        mesh=_MESH,
        scratch_types=[
            pltpu.VMEM((2, W), jnp.int32),
            pltpu.VMEM((2, max(tail, 8)), jnp.int32),
            pltpu.VMEM((W, F), jnp.float32),
            pltpu.VMEM((max(tail, 8), F), jnp.float32),
            pltpu.VMEM((RPT // 5, F), jnp.float32),
            pltpu.VMEM_SHARED((NP, F), jnp.float32),
            pltpu.SemaphoreType.DMA,
        ],
        name=f"prop_{src_row}{dst_row}{int(feat_split)}",
    )


_MB = 2000

def _recip(cnt):
    return jnp.where(cnt > 0, 1.0 / cnt, 0.0)


def _scale_sum_body(p_ref, cnt_ref, o_ref):
    o_ref[...] = (p_ref[0] + p_ref[1]) * _recip(cnt_ref[...])


def _scale_sum(p, cnt):
    return pl.pallas_call(
        _scale_sum_body,
        grid=(N // _MB,),
        in_specs=[
            pl.BlockSpec((NC, _MB, F), lambda m: (0, m, 0)),
            pl.BlockSpec((_MB, 1), lambda m: (m, 0)),
        ],
        out_specs=pl.BlockSpec((_MB, F), lambda m: (m, 0)),
        out_shape=jax.ShapeDtypeStruct((N, F), jnp.float32),
    )(p, cnt)


def _scale_body(e_ref, cnt_ref, o_ref):
    o_ref[...] = e_ref[...] * _recip(cnt_ref[...])


def _scale(e, cnt):
    return pl.pallas_call(
        _scale_body,
        grid=(NC, N // _MB),
        in_specs=[
            pl.BlockSpec((1, _MB, F), lambda h, m: (h, m, 0)),
            pl.BlockSpec((_MB, 1), lambda h, m: (m, 0)),
        ],
        out_specs=pl.BlockSpec((1, _MB, F), lambda h, m: (h, m, 0)),
        out_shape=jax.ShapeDtypeStruct((NC, N, F), jnp.float32),
    )(e, cnt)


def _leaky(y):
    return jnp.where(y > 0, y, 0.01 * y)


def _mm1_body(t_ref, cnt_ref, w_ref, b_ref, o_ref):
    x = (t_ref[0] + t_ref[1]) * _recip(cnt_ref[...])
    y = lax.dot_general(x, w_ref[0], (((1,), (1,)), ((), ())),
                        precision=lax.Precision.HIGHEST)
    o_ref[...] = _leaky(y + b_ref[0])[None]


def _mm1(t, cnt, w_split, b_split):
    return pl.pallas_call(
        _mm1_body,
        grid=(NC, N // _MB),
        in_specs=[
            pl.BlockSpec((NC, _MB, F), lambda h, m: (0, m, 0)),
            pl.BlockSpec((_MB, 1), lambda h, m: (m, 0)),
            pl.BlockSpec((1, F, F), lambda h, m: (h, 0, 0)),
            pl.BlockSpec((1, 1, F), lambda h, m: (h, 0, 0)),
        ],
        out_specs=pl.BlockSpec((1, _MB, F), lambda h, m: (h, m, 0)),
        out_shape=jax.ShapeDtypeStruct((NC, N, F), jnp.float32),
    )(t, cnt, w_split, b_split)


def _mm23_body(t_ref, cnt_ref, w2_ref, b2_ref, w3_ref, o_ref):
    dinv = _recip(cnt_ref[...])
    y = lax.dot_general(t_ref[0] * dinv, w2_ref[0], (((1,), (1,)), ((), ())),
                        precision=lax.Precision.HIGHEST)
    y += lax.dot_general(t_ref[1] * dinv, w2_ref[1], (((1,), (1,)), ((), ())),
                         precision=lax.Precision.HIGHEST)
    h2 = _leaky(y + b2_ref[...])
    o_ref[...] = lax.dot_general(h2, w3_ref[...], (((1,), (1,)), ((), ())),
                                 precision=lax.Precision.HIGHEST)


def _mm23(t, cnt, w2_split, b2, w3):
    return pl.pallas_call(
        _mm23_body,
        grid=(N // _MB,),
        in_specs=[
            pl.BlockSpec((NC, _MB, F), lambda m: (0, m, 0)),
            pl.BlockSpec((_MB, 1), lambda m: (m, 0)),
            pl.BlockSpec((NC, 2 * F, F), lambda m: (0, 0, 0)),
            pl.BlockSpec((1, 2 * F), lambda m: (0, 0)),
            pl.BlockSpec((F, 2 * F), lambda m: (0, 0)),
        ],
        out_specs=pl.BlockSpec((_MB, F), lambda m: (m, 0)),
        out_shape=jax.ShapeDtypeStruct((N, F), jnp.float32),
    )(t, cnt, w2_split, b2, w3)


def _pool_body(t_ref, cnt_ref, b3_ref, wa_ref, o_ref):
    h3 = (t_ref[0] + t_ref[1]) * _recip(cnt_ref[...]) + b3_ref[...]
    s = lax.dot_general(h3, wa_ref[...], (((1,), (1,)), ((), ())),
                        precision=lax.Precision.HIGHEST)
    w = jnp.exp(s - jnp.max(s))
    o_ref[...] = jnp.sum(w * h3, axis=0, keepdims=True) / jnp.sum(w)


def _pool(t, cnt, b3, wa):
    return pl.pallas_call(
        _pool_body,
        grid=(1,),
        in_specs=[
            pl.BlockSpec((NC, N, F), lambda i: (0, 0, 0)),
            pl.BlockSpec((N, 1), lambda i: (0, 0)),
            pl.BlockSpec((1, F), lambda i: (0, 0)),
            pl.BlockSpec((1, F), lambda i: (0, 0)),
        ],
        out_specs=pl.BlockSpec((1, F), lambda i: (0, 0)),
        out_shape=jax.ShapeDtypeStruct((1, F), jnp.float32),
    )(t, cnt, b3, wa)


def kernel(x, hyper_edge_index, W1, b1, W2, b2, W3, b3, Wa, ba):
    del ba
    idx = hyper_edge_index.astype(jnp.int32).reshape(2 * NNZ)

    counts = _counts(idx)
    cnt_n = counts[0, :N, None]
    cnt_e = counts[1, :N, None]

    w1s = W1.reshape(NC, F, F)
    w2s = jnp.stack([W2[:, :F], W2[:, F:]])
    b1s = b1.reshape(NC, 1, F)

    prop_ne_n = _make_prop(0, 1, False)
    prop_en_n = _make_prop(1, 0, False)
    prop_ne_f = _make_prop(0, 1, True)
    prop_en_f = _make_prop(1, 0, True)

    p1 = prop_ne_n(x, idx)
    e1 = _scale_sum(p1, cnt_e)
    t1 = prop_en_n(e1, idx)
    h1 = _mm1(t1, cnt_n, w1s, b1s)

    e2 = _scale(prop_ne_f(h1, idx), cnt_e)
    t2 = prop_en_f(e2, idx)
    g3 = _mm23(t2, cnt_n, w2s, b2[None], W3)

    p3 = prop_ne_n(g3, idx)
    e3 = _scale_sum(p3, cnt_e)
    t3 = prop_en_n(e3, idx)

    out = _pool(t3, cnt_n, b3[None], Wa)
    return out[0]


# speedup vs baseline: 19.1850x; 2.2929x over previous
import functools

import jax
import jax.numpy as jnp
from jax import lax
from jax.experimental import pallas as pl
from jax.experimental.pallas import tpu as pltpu
from jax.experimental.pallas import tpu_sc as plsc

N = 10000
E = 10000
NNZ = 320000
F = 128
NC = 2
NS = 16
W = 128
NP = 10240
RPT = NP // NS

_MESH = plsc.VectorSubcoreMesh(
    core_axis_name="c", subcore_axis_name="s", num_cores=NC, num_subcores=NS)


def _zero_vmem(buf, rows, cols):
    zero = jnp.zeros((16,), jnp.float32)

    def _row(r, _):
        def _col(j, _):
            buf[r, pl.ds(j * 16, 16)] = zero
            return 0
        return lax.fori_loop(0, cols // 16, _col, 0)

    lax.fori_loop(0, rows, _row, 0)


NWALL = NNZ // W
NWPAD = 2560
NWT_F = NWPAD // NS
NWT_N = NWPAD // (NC * NS)


def _counts_body(idx_hbm, out_hbm, ib, ones_v, zb, acc, sem):
    c = lax.axis_index("c")
    s = lax.axis_index("s")

    def _zo(j, _):
        zb[pl.ds(j * 16, 16)] = jnp.zeros((16,), jnp.float32)
        return 0
    lax.fori_loop(0, RPT // 16, _zo, 0)
    pltpu.sync_copy(zb, acc.at[pl.ds(s * RPT, RPT)])

    def _one(j, _):
        ones_v[pl.ds(j * 16, 16)] = jnp.ones((16,), jnp.float32)
        return 0
    lax.fori_loop(0, W // 16, _one, 0)

    base = c * NNZ + s * 156 * W
    nw = jnp.where(s == NS - 1, 160, 156)

    def _stage(w, _):
        pltpu.async_copy(idx_hbm.at[pl.ds(base + w * W, W)], ib.at[w], sem)
        return 0
    lax.fori_loop(0, nw, _stage, 0)

    def _sdrain(w, _):
        pltpu.make_async_copy(idx_hbm.at[pl.ds(base + w * W, W)],
                              ib.at[w], sem).wait()
        return 0
    lax.fori_loop(0, nw, _sdrain, 0)

    plsc.subcore_barrier()

    def _win(g, _):
        pltpu.async_copy(ones_v, acc.at[ib.at[g]], sem, add=True)
        return 0
    lax.fori_loop(0, nw, _win, 0)

    def _drain(g, _):
        pltpu.make_async_copy(ones_v, acc.at[ib.at[g]], sem).wait()
        return 0
    lax.fori_loop(0, nw, _drain, 0)

    plsc.subcore_barrier()
    pltpu.sync_copy(acc.at[pl.ds(s * RPT, RPT)],
                    out_hbm.at[c, pl.ds(s * RPT, RPT)])


_counts = pl.kernel(
    _counts_body,
    out_type=jax.ShapeDtypeStruct((NC, NP), jnp.float32),
    mesh=_MESH,
    scratch_types=[
        pltpu.VMEM((160, W), jnp.int32),
        pltpu.VMEM((W,), jnp.float32),
        pltpu.VMEM((RPT,), jnp.float32),
        pltpu.VMEM_SHARED((NP,), jnp.float32),
        pltpu.SemaphoreType.DMA,
    ],
)


def _prop_body(src_row, dst_row, feat_split, v_hbm, idx_hbm, out_hbm,
               ib, rows, acc, isem, gsem, ssem):
    c = lax.axis_index("c")
    s = lax.axis_index("s")

    _zero_vmem(rows, W, F)
    for k in range(5):
        pltpu.sync_copy(rows.at[pl.ds(0, W)],
                        acc.at[pl.ds(s * RPT + k * (RPT // 5), RPT // 5)])

    if feat_split:
        wpt = NWALL // NS
        base = s * wpt * W
        nw = jnp.where(s == NS - 1, wpt + 4, wpt)
        vsrc = v_hbm.at[c]
    else:
        wpt = NWALL // (NC * NS)
        wid = c * NS + s
        base = wid * wpt * W
        nw = jnp.where(wid == NC * NS - 1, wpt + 4, wpt)
        vsrc = v_hbm

    soff = src_row * NNZ + base
    doff = dst_row * NNZ + base

    def _idesc(g):
        islot = lax.rem(g, 3)
        src_i = idx_hbm.at[pl.ds(soff + g * W, W)]
        dst_i = idx_hbm.at[pl.ds(doff + g * W, W)]
        return (src_i, ib.at[0, islot], dst_i, ib.at[1, islot], islot)

    def _issue_i(g):
        src_i, d0, dst_i, d1, islot = _idesc(g)
        pltpu.async_copy(src_i, d0, isem.at[islot])
        pltpu.async_copy(dst_i, d1, isem.at[islot])

    def _wait_i(g):
        src_i, d0, dst_i, d1, islot = _idesc(g)
        pltpu.make_async_copy(src_i, d0, isem.at[islot]).wait()
        pltpu.make_async_copy(dst_i, d1, isem.at[islot]).wait()

    def _gdesc(g):
        slot = lax.rem(g, 2)
        roff = pl.multiple_of(slot * W, W)
        return (vsrc.at[ib.at[0, lax.rem(g, 3)]], rows.at[pl.ds(roff, W)],
                slot)

    def _issue_g(g):
        src, dst, slot = _gdesc(g)
        pltpu.async_copy(src, dst, gsem.at[slot])

    def _wait_g(g):
        src, dst, slot = _gdesc(g)
        pltpu.make_async_copy(src, dst, gsem.at[slot]).wait()

    def _sdesc(g):
        slot = lax.rem(g, 2)
        roff = pl.multiple_of(slot * W, W)
        return (rows.at[pl.ds(roff, W)], acc.at[ib.at[1, lax.rem(g, 3)]],
                slot)

    def _issue_s(g):
        src, dst, slot = _sdesc(g)
        pltpu.async_copy(src, dst, ssem.at[slot], add=True)

    def _wait_s(g):
        src, dst, slot = _sdesc(g)
        pltpu.make_async_copy(src, dst, ssem.at[slot]).wait()

    plsc.subcore_barrier()

    _issue_i(jnp.int32(0))
    _issue_i(jnp.int32(1))
    _wait_i(jnp.int32(0))
    _issue_g(jnp.int32(0))

    def _win(g, _):
        @pl.when(g >= 1)
        def _():
            _wait_s(g - 1)

        @pl.when(g + 2 < nw)
        def _():
            _issue_i(g + 2)

        @pl.when(g + 1 < nw)
        def _():
            _wait_i(g + 1)
            _issue_g(g + 1)

        _wait_g(g)
        _issue_s(g)
        return 0
    lax.fori_loop(0, nw, _win, 0)
    _wait_s(nw - 1)

    plsc.subcore_barrier()
    pltpu.sync_copy(acc.at[pl.ds(s * RPT, RPT)],
                    out_hbm.at[c, pl.ds(s * RPT, RPT)])


@functools.cache
def _make_prop(src_row, dst_row, feat_split, tag=0):
    return pl.kernel(
        functools.partial(_prop_body, src_row, dst_row, feat_split),
        out_type=jax.ShapeDtypeStruct((NC, NP, F), jnp.float32),
        mesh=_MESH,
        scratch_types=[
            pltpu.VMEM((2, 3, W), jnp.int32),
            pltpu.VMEM((2 * W, F), jnp.float32),
            pltpu.VMEM_SHARED((NP, F), jnp.float32),
            pltpu.SemaphoreType.DMA((3,)),
            pltpu.SemaphoreType.DMA((2,)),
            pltpu.SemaphoreType.DMA((2,)),
        ],
        name=f"prop_{src_row}{dst_row}{int(feat_split)}_{tag}",
    )


_MB = 2000


def _recip(cnt):
    return jnp.where(cnt > 0, 1.0 / cnt, 0.0)


def _scale_sum_body(p_ref, cnt_ref, o_ref):
    o_ref[...] = (p_ref[0] + p_ref[1]) * _recip(cnt_ref[...])


def _scale_sum(p, cnt):
    return pl.pallas_call(
        _scale_sum_body,
        grid=(N // _MB,),
        in_specs=[
            pl.BlockSpec((NC, _MB, F), lambda m: (0, m, 0)),
            pl.BlockSpec((_MB, 1), lambda m: (m, 0)),
        ],
        out_specs=pl.BlockSpec((_MB, F), lambda m: (m, 0)),
        out_shape=jax.ShapeDtypeStruct((N, F), jnp.float32),
    )(p, cnt)


def _scale_body(e_ref, cnt_ref, o_ref):
    o_ref[...] = e_ref[...] * _recip(cnt_ref[...])


def _scale(e, cnt):
    return pl.pallas_call(
        _scale_body,
        grid=(NC, N // _MB),
        in_specs=[
            pl.BlockSpec((1, _MB, F), lambda h, m: (h, m, 0)),
            pl.BlockSpec((_MB, 1), lambda h, m: (m, 0)),
        ],
        out_specs=pl.BlockSpec((1, _MB, F), lambda h, m: (h, m, 0)),
        out_shape=jax.ShapeDtypeStruct((NC, N, F), jnp.float32),
    )(e, cnt)


def _leaky(y):
    return jnp.where(y > 0, y, 0.01 * y)


def _mm1_body(t_ref, cnt_ref, w_ref, b_ref, o_ref):
    x = (t_ref[0] + t_ref[1]) * _recip(cnt_ref[...])
    y = lax.dot_general(x, w_ref[0], (((1,), (1,)), ((), ())),
                        precision=lax.Precision.HIGHEST)
    o_ref[...] = _leaky(y + b_ref[0])[None]


def _mm1(t, cnt, w_split, b_split):
    return pl.pallas_call(
        _mm1_body,
        grid=(NC, N // _MB),
        in_specs=[
            pl.BlockSpec((NC, _MB, F), lambda h, m: (0, m, 0)),
            pl.BlockSpec((_MB, 1), lambda h, m: (m, 0)),
            pl.BlockSpec((1, F, F), lambda h, m: (h, 0, 0)),
            pl.BlockSpec((1, 1, F), lambda h, m: (h, 0, 0)),
        ],
        out_specs=pl.BlockSpec((1, _MB, F), lambda h, m: (h, m, 0)),
        out_shape=jax.ShapeDtypeStruct((NC, N, F), jnp.float32),
    )(t, cnt, w_split, b_split)


def _mm23_body(t_ref, cnt_ref, w2_ref, b2_ref, w3_ref, o_ref):
    dinv = _recip(cnt_ref[...])
    y = lax.dot_general(t_ref[0] * dinv, w2_ref[0], (((1,), (1,)), ((), ())),
                        precision=lax.Precision.HIGHEST)
    y += lax.dot_general(t_ref[1] * dinv, w2_ref[1], (((1,), (1,)), ((), ())),
                         precision=lax.Precision.HIGHEST)
    h2 = _leaky(y + b2_ref[...])
    o_ref[...] = lax.dot_general(h2, w3_ref[...], (((1,), (1,)), ((), ())),
                                 precision=lax.Precision.HIGHEST)


def _mm23(t, cnt, w2_split, b2, w3):
    return pl.pallas_call(
        _mm23_body,
        grid=(N // _MB,),
        in_specs=[
            pl.BlockSpec((NC, _MB, F), lambda m: (0, m, 0)),
            pl.BlockSpec((_MB, 1), lambda m: (m, 0)),
            pl.BlockSpec((NC, 2 * F, F), lambda m: (0, 0, 0)),
            pl.BlockSpec((1, 2 * F), lambda m: (0, 0)),
            pl.BlockSpec((F, 2 * F), lambda m: (0, 0)),
        ],
        out_specs=pl.BlockSpec((_MB, F), lambda m: (m, 0)),
        out_shape=jax.ShapeDtypeStruct((N, F), jnp.float32),
    )(t, cnt, w2_split, b2, w3)


def _pool_body(t_ref, cnt_ref, b3_ref, wa_ref, o_ref):
    h3 = (t_ref[0] + t_ref[1]) * _recip(cnt_ref[...]) + b3_ref[...]
    s = lax.dot_general(h3, wa_ref[...], (((1,), (1,)), ((), ())),
                        precision=lax.Precision.HIGHEST)
    w = jnp.exp(s - jnp.max(s))
    o_ref[...] = jnp.sum(w * h3, axis=0, keepdims=True) / jnp.sum(w)


def _pool(t, cnt, b3, wa):
    return pl.pallas_call(
        _pool_body,
        grid=(1,),
        in_specs=[
            pl.BlockSpec((NC, N, F), lambda i: (0, 0, 0)),
            pl.BlockSpec((N, 1), lambda i: (0, 0)),
            pl.BlockSpec((1, F), lambda i: (0, 0)),
            pl.BlockSpec((1, F), lambda i: (0, 0)),
        ],
        out_specs=pl.BlockSpec((1, F), lambda i: (0, 0)),
        out_shape=jax.ShapeDtypeStruct((1, F), jnp.float32),
    )(t, cnt, b3, wa)


def kernel(x, hyper_edge_index, W1, b1, W2, b2, W3, b3, Wa, ba):
    del ba
    idx = hyper_edge_index.astype(jnp.int32).reshape(2 * NNZ)

    counts = _counts(idx)
    cnt_n = counts[0, :N, None]
    cnt_e = counts[1, :N, None]

    w1s = W1.reshape(NC, F, F)
    w2s = jnp.stack([W2[:, :F], W2[:, F:]])
    b1s = b1.reshape(NC, 1, F)

    prop_ne_n = _make_prop(0, 1, False)
    prop_en_n = _make_prop(1, 0, False)
    prop_ne_f = _make_prop(0, 1, True)
    prop_en_f = _make_prop(1, 0, True)

    p1 = prop_ne_n(x, idx)
    e1 = _scale_sum(p1, cnt_e)
    t1 = prop_en_n(e1, idx)
    h1 = _mm1(t1, cnt_n, w1s, b1s)

    e2 = _scale(prop_ne_f(h1, idx), cnt_e)
    t2 = prop_en_f(e2, idx)
    g3 = _mm23(t2, cnt_n, w2s, b2[None], W3)

    p3 = _make_prop(0, 1, False, tag=1)(g3, idx)
    e3 = _scale_sum(p3, cnt_e)
    t3 = _make_prop(1, 0, False, tag=1)(e3, idx)

    out = _pool(t3, cnt_n, b3[None], Wa)
    return out[0]


# speedup vs baseline: 21.1519x; 1.1025x over previous
import functools

import jax
import jax.numpy as jnp
from jax import lax
from jax.experimental import pallas as pl
from jax.experimental.pallas import tpu as pltpu
from jax.experimental.pallas import tpu_sc as plsc

N = 10000
E = 10000
NNZ = 320000
F = 128
NC = 2
NS = 16
W = 128
NP = 10240
RPT = NP // NS

_MESH = plsc.VectorSubcoreMesh(
    core_axis_name="c", subcore_axis_name="s", num_cores=NC, num_subcores=NS)


def _zero_vmem(buf, rows, cols):
    zero = jnp.zeros((16,), jnp.float32)

    def _row(r, _):
        def _col(j, _):
            buf[r, pl.ds(j * 16, 16)] = zero
            return 0
        return lax.fori_loop(0, cols // 16, _col, 0)

    lax.fori_loop(0, rows, _row, 0)


NWALL = NNZ // W
NWPAD = 2560
NWT_F = NWPAD // NS
NWT_N = NWPAD // (NC * NS)


def _counts_body(idx_hbm, out_hbm, ib, ones_v, zb, acc, sem):
    c = lax.axis_index("c")
    s = lax.axis_index("s")

    def _zo(j, _):
        zb[pl.ds(j * 16, 16)] = jnp.zeros((16,), jnp.float32)
        return 0
    lax.fori_loop(0, RPT // 16, _zo, 0)
    pltpu.sync_copy(zb, acc.at[pl.ds(s * RPT, RPT)])

    def _one(j, _):
        ones_v[pl.ds(j * 16, 16)] = jnp.ones((16,), jnp.float32)
        return 0
    lax.fori_loop(0, W // 16, _one, 0)

    base = c * NNZ + s * 156 * W
    nw = jnp.where(s == NS - 1, 160, 156)

    def _stage(w, _):
        pltpu.async_copy(idx_hbm.at[pl.ds(base + w * W, W)], ib.at[w], sem)
        return 0
    lax.fori_loop(0, nw, _stage, 0)

    def _sdrain(w, _):
        pltpu.make_async_copy(idx_hbm.at[pl.ds(base + w * W, W)],
                              ib.at[w], sem).wait()
        return 0
    lax.fori_loop(0, nw, _sdrain, 0)

    plsc.subcore_barrier()

    def _win(g, _):
        pltpu.async_copy(ones_v, acc.at[ib.at[g]], sem, add=True)
        return 0
    lax.fori_loop(0, nw, _win, 0)

    def _drain(g, _):
        pltpu.make_async_copy(ones_v, acc.at[ib.at[g]], sem).wait()
        return 0
    lax.fori_loop(0, nw, _drain, 0)

    plsc.subcore_barrier()
    pltpu.sync_copy(acc.at[pl.ds(s * RPT, RPT)],
                    out_hbm.at[c, pl.ds(s * RPT, RPT)])


_counts = pl.kernel(
    _counts_body,
    out_type=jax.ShapeDtypeStruct((NC, NP), jnp.float32),
    mesh=_MESH,
    scratch_types=[
        pltpu.VMEM((160, W), jnp.int32),
        pltpu.VMEM((W,), jnp.float32),
        pltpu.VMEM((RPT,), jnp.float32),
        pltpu.VMEM_SHARED((NP,), jnp.float32),
        pltpu.SemaphoreType.DMA,
    ],
)


RPP = 624


def _prop_body(src_row, dst_row, feat_split, v_hbm, idx_hbm, out_hbm,
               ib, rows, acc, isem, gsem, ssem):
    c = lax.axis_index("c")
    s = lax.axis_index("s")

    _zero_vmem(rows, 208, F)
    for k in range(3):
        pltpu.sync_copy(rows.at[pl.ds(0, 208)],
                        acc.at[pl.ds(s * RPP + k * 208, 208)])

    @pl.when(s == NS - 1)
    def _():
        pltpu.sync_copy(rows.at[pl.ds(0, 16)], acc.at[pl.ds(N - 16, 16)])

    if feat_split:
        wpt = NWALL // NS
        base = s * wpt * W
        nw = jnp.where(s == NS - 1, wpt + 4, wpt)
        vsrc = v_hbm.at[c]
    else:
        wpt = NWALL // (NC * NS)
        wid = c * NS + s
        base = wid * wpt * W
        nw = jnp.where(wid == NC * NS - 1, wpt + 4, wpt)
        vsrc = v_hbm

    soff = src_row * NNZ + base
    doff = dst_row * NNZ + base

    def _idesc(g):
        islot = lax.rem(g, 4)
        src_i = idx_hbm.at[pl.ds(soff + g * W, W)]
        dst_i = idx_hbm.at[pl.ds(doff + g * W, W)]
        return (src_i, ib.at[0, islot], dst_i, ib.at[1, islot], islot)

    def _issue_i(g):
        src_i, d0, dst_i, d1, islot = _idesc(g)
        pltpu.async_copy(src_i, d0, isem.at[islot])
        pltpu.async_copy(dst_i, d1, isem.at[islot])

    def _wait_i(g):
        src_i, d0, dst_i, d1, islot = _idesc(g)
        pltpu.make_async_copy(src_i, d0, isem.at[islot]).wait()
        pltpu.make_async_copy(dst_i, d1, isem.at[islot]).wait()

    def _gdesc(g):
        slot = lax.rem(g, 3)
        roff = pl.multiple_of(slot * W, W)
        return (vsrc.at[ib.at[0, lax.rem(g, 4)]], rows.at[pl.ds(roff, W)],
                slot)

    def _issue_g(g):
        src, dst, gslot = _gdesc(g)
        pltpu.async_copy(src, dst, gsem.at[gslot])

    def _wait_g(g):
        src, dst, gslot = _gdesc(g)
        pltpu.make_async_copy(src, dst, gsem.at[gslot]).wait()

    def _sdesc(g):
        slot = lax.rem(g, 3)
        roff = pl.multiple_of(slot * W, W)
        return (rows.at[pl.ds(roff, W)], acc.at[ib.at[1, lax.rem(g, 4)]])

    def _issue_s(g):
        src, dst = _sdesc(g)
        pltpu.async_copy(src, dst, ssem, add=True)

    def _wait_s(g):
        src, dst = _sdesc(g)
        pltpu.make_async_copy(src, dst, ssem).wait()

    plsc.subcore_barrier()

    _issue_i(jnp.int32(0))
    _issue_i(jnp.int32(1))
    _issue_i(jnp.int32(2))
    _wait_i(jnp.int32(0))
    _issue_g(jnp.int32(0))
    _wait_i(jnp.int32(1))
    _issue_g(jnp.int32(1))

    def _win(g, _):
        @pl.when(g >= 1)
        def _():
            _wait_s(g - 1)

        @pl.when(g + 3 < nw)
        def _():
            _issue_i(g + 3)

        @pl.when(g + 2 < nw)
        def _():
            _wait_i(g + 2)
            _issue_g(g + 2)

        _wait_g(g)
        _issue_s(g)
        return 0
    lax.fori_loop(0, nw, _win, 0)
    _wait_s(nw - 1)

    plsc.subcore_barrier()

    @pl.when(s < NS - 1)
    def _():
        pltpu.sync_copy(acc.at[pl.ds(s * RPP, RPP)],
                        out_hbm.at[c, pl.ds(s * RPP, RPP)])

    @pl.when(s == NS - 1)
    def _():
        pltpu.sync_copy(acc.at[pl.ds((NS - 1) * RPP, 640)],
                        out_hbm.at[c, pl.ds((NS - 1) * RPP, 640)])


@functools.cache
def _make_prop(src_row, dst_row, feat_split, tag=0):
    return pl.kernel(
        functools.partial(_prop_body, src_row, dst_row, feat_split),
        out_type=jax.ShapeDtypeStruct((NC, N, F), jnp.float32),
        mesh=_MESH,
        scratch_types=[
            pltpu.VMEM((2, 4, W), jnp.int32),
            pltpu.VMEM((3 * W, F), jnp.float32),
            pltpu.VMEM_SHARED((N, F), jnp.float32),
            pltpu.SemaphoreType.DMA((4,)),
            pltpu.SemaphoreType.DMA((3,)),
            pltpu.SemaphoreType.DMA,
        ],
        name=f"prop_{src_row}{dst_row}{int(feat_split)}_{tag}",
    )


_MB = 2000


def _recip(cnt):
    return jnp.where(cnt > 0, 1.0 / cnt, 0.0)


def _scale_sum_body(p_ref, cnt_ref, o_ref):
    o_ref[...] = (p_ref[0] + p_ref[1]) * _recip(cnt_ref[...])


def _scale_sum(p, cnt):
    return pl.pallas_call(
        _scale_sum_body,
        grid=(N // _MB,),
        in_specs=[
            pl.BlockSpec((NC, _MB, F), lambda m: (0, m, 0)),
            pl.BlockSpec((_MB, 1), lambda m: (m, 0)),
        ],
        out_specs=pl.BlockSpec((_MB, F), lambda m: (m, 0)),
        out_shape=jax.ShapeDtypeStruct((N, F), jnp.float32),
    )(p, cnt)


def _scale_body(e_ref, cnt_ref, o_ref):
    o_ref[...] = e_ref[...] * _recip(cnt_ref[...])


def _scale(e, cnt):
    return pl.pallas_call(
        _scale_body,
        grid=(NC, N // _MB),
        in_specs=[
            pl.BlockSpec((1, _MB, F), lambda h, m: (h, m, 0)),
            pl.BlockSpec((_MB, 1), lambda h, m: (m, 0)),
        ],
        out_specs=pl.BlockSpec((1, _MB, F), lambda h, m: (h, m, 0)),
        out_shape=jax.ShapeDtypeStruct((NC, N, F), jnp.float32),
    )(e, cnt)


def _leaky(y):
    return jnp.where(y > 0, y, 0.01 * y)


def _mm1_body(t_ref, cnt_ref, w_ref, b_ref, o_ref):
    x = (t_ref[0] + t_ref[1]) * _recip(cnt_ref[...])
    y = lax.dot_general(x, w_ref[0], (((1,), (1,)), ((), ())),
                        precision=lax.Precision.HIGHEST)
    o_ref[...] = _leaky(y + b_ref[0])[None]


def _mm1(t, cnt, w_split, b_split):
    return pl.pallas_call(
        _mm1_body,
        grid=(NC, N // _MB),
        in_specs=[
            pl.BlockSpec((NC, _MB, F), lambda h, m: (0, m, 0)),
            pl.BlockSpec((_MB, 1), lambda h, m: (m, 0)),
            pl.BlockSpec((1, F, F), lambda h, m: (h, 0, 0)),
            pl.BlockSpec((1, 1, F), lambda h, m: (h, 0, 0)),
        ],
        out_specs=pl.BlockSpec((1, _MB, F), lambda h, m: (h, m, 0)),
        out_shape=jax.ShapeDtypeStruct((NC, N, F), jnp.float32),
    )(t, cnt, w_split, b_split)


def _mm23_body(t_ref, cnt_ref, w2_ref, b2_ref, w3_ref, o_ref):
    dinv = _recip(cnt_ref[...])
    y = lax.dot_general(t_ref[0] * dinv, w2_ref[0], (((1,), (1,)), ((), ())),
                        precision=lax.Precision.HIGHEST)
    y += lax.dot_general(t_ref[1] * dinv, w2_ref[1], (((1,), (1,)), ((), ())),
                         precision=lax.Precision.HIGHEST)
    h2 = _leaky(y + b2_ref[...])
    o_ref[...] = lax.dot_general(h2, w3_ref[...], (((1,), (1,)), ((), ())),
                                 precision=lax.Precision.HIGHEST)


def _mm23(t, cnt, w2_split, b2, w3):
    return pl.pallas_call(
        _mm23_body,
        grid=(N // _MB,),
        in_specs=[
            pl.BlockSpec((NC, _MB, F), lambda m: (0, m, 0)),
            pl.BlockSpec((_MB, 1), lambda m: (m, 0)),
            pl.BlockSpec((NC, 2 * F, F), lambda m: (0, 0, 0)),
            pl.BlockSpec((1, 2 * F), lambda m: (0, 0)),
            pl.BlockSpec((F, 2 * F), lambda m: (0, 0)),
        ],
        out_specs=pl.BlockSpec((_MB, F), lambda m: (m, 0)),
        out_shape=jax.ShapeDtypeStruct((N, F), jnp.float32),
    )(t, cnt, w2_split, b2, w3)


def _pool_body(t_ref, cnt_ref, b3_ref, wa_ref, o_ref):
    h3 = (t_ref[0] + t_ref[1]) * _recip(cnt_ref[...]) + b3_ref[...]
    s = lax.dot_general(h3, wa_ref[...], (((1,), (1,)), ((), ())),
                        precision=lax.Precision.HIGHEST)
    w = jnp.exp(s - jnp.max(s))
    o_ref[...] = jnp.sum(w * h3, axis=0, keepdims=True) / jnp.sum(w)


def _pool(t, cnt, b3, wa):
    return pl.pallas_call(
        _pool_body,
        grid=(1,),
        in_specs=[
            pl.BlockSpec((NC, N, F), lambda i: (0, 0, 0)),
            pl.BlockSpec((N, 1), lambda i: (0, 0)),
            pl.BlockSpec((1, F), lambda i: (0, 0)),
            pl.BlockSpec((1, F), lambda i: (0, 0)),
        ],
        out_specs=pl.BlockSpec((1, F), lambda i: (0, 0)),
        out_shape=jax.ShapeDtypeStruct((1, F), jnp.float32),
    )(t, cnt, b3, wa)


def kernel(x, hyper_edge_index, W1, b1, W2, b2, W3, b3, Wa, ba):
    del ba
    idx = hyper_edge_index.astype(jnp.int32).reshape(2 * NNZ)

    counts = _counts(idx)
    cnt_n = counts[0, :N, None]
    cnt_e = counts[1, :N, None]

    w1s = W1.reshape(NC, F, F)
    w2s = jnp.stack([W2[:, :F], W2[:, F:]])
    b1s = b1.reshape(NC, 1, F)

    prop_ne_n = _make_prop(0, 1, False)
    prop_en_n = _make_prop(1, 0, False)
    prop_ne_f = _make_prop(0, 1, True)
    prop_en_f = _make_prop(1, 0, True)

    p1 = prop_ne_n(x, idx)
    e1 = _scale_sum(p1, cnt_e)
    t1 = prop_en_n(e1, idx)
    h1 = _mm1(t1, cnt_n, w1s, b1s)

    e2 = _scale(prop_ne_f(h1, idx), cnt_e)
    t2 = prop_en_f(e2, idx)
    g3 = _mm23(t2, cnt_n, w2s, b2[None], W3)

    p3 = _make_prop(0, 1, False, tag=1)(g3, idx)
    e3 = _scale_sum(p3, cnt_e)
    t3 = _make_prop(1, 0, False, tag=1)(e3, idx)

    out = _pool(t3, cnt_n, b3[None], Wa)
    return out[0]


# speedup vs baseline: 21.1614x; 1.0005x over previous
import functools

import jax
import jax.numpy as jnp
from jax import lax
from jax.experimental import pallas as pl
from jax.experimental.pallas import tpu as pltpu
from jax.experimental.pallas import tpu_sc as plsc

N = 10000
E = 10000
NNZ = 320000
F = 128
NC = 2
NS = 16
W = 128
NP = 10240
RPT = NP // NS

_MESH = plsc.VectorSubcoreMesh(
    core_axis_name="c", subcore_axis_name="s", num_cores=NC, num_subcores=NS)


def _zero_vmem(buf, rows, cols):
    zero = jnp.zeros((16,), jnp.float32)

    def _row(r, _):
        def _col(j, _):
            buf[r, pl.ds(j * 16, 16)] = zero
            return 0
        return lax.fori_loop(0, cols // 16, _col, 0)

    lax.fori_loop(0, rows, _row, 0)


NWALL = NNZ // W


def _counts_body(idx_hbm, out_hbm, ib, ones_v, zb, acc, sem):
    c = lax.axis_index("c")
    s = lax.axis_index("s")

    def _zo(j, _):
        zb[pl.ds(j * 16, 16)] = jnp.zeros((16,), jnp.float32)
        return 0
    lax.fori_loop(0, RPT // 16, _zo, 0)
    pltpu.sync_copy(zb, acc.at[pl.ds(s * RPT, RPT)])

    def _one(j, _):
        ones_v[pl.ds(j * 16, 16)] = jnp.ones((16,), jnp.float32)
        return 0
    lax.fori_loop(0, W // 16, _one, 0)

    base = c * NNZ + s * 156 * W
    nw = jnp.where(s == NS - 1, 160, 156)

    def _stage(w, _):
        pltpu.async_copy(idx_hbm.at[pl.ds(base + w * W, W)], ib.at[w], sem)
        return 0
    lax.fori_loop(0, nw, _stage, 0)

    def _sdrain(w, _):
        pltpu.make_async_copy(idx_hbm.at[pl.ds(base + w * W, W)],
                              ib.at[w], sem).wait()
        return 0
    lax.fori_loop(0, nw, _sdrain, 0)

    plsc.subcore_barrier()

    def _win(g, _):
        pltpu.async_copy(ones_v, acc.at[ib.at[g]], sem, add=True)
        return 0
    lax.fori_loop(0, nw, _win, 0)

    def _drain(g, _):
        pltpu.make_async_copy(ones_v, acc.at[ib.at[g]], sem).wait()
        return 0
    lax.fori_loop(0, nw, _drain, 0)

    plsc.subcore_barrier()
    pltpu.sync_copy(acc.at[pl.ds(s * RPT, RPT)],
                    out_hbm.at[c, pl.ds(s * RPT, RPT)])


_counts = pl.kernel(
    _counts_body,
    out_type=jax.ShapeDtypeStruct((NC, NP), jnp.float32),
    mesh=_MESH,
    scratch_types=[
        pltpu.VMEM((160, W), jnp.int32),
        pltpu.VMEM((W,), jnp.float32),
        pltpu.VMEM((RPT,), jnp.float32),
        pltpu.VMEM_SHARED((NP,), jnp.float32),
        pltpu.SemaphoreType.DMA,
    ],
)


RPP = 624


def _prop_body(src_row, dst_row, feat_split, v_hbm, idx_hbm, out_hbm,
               ib, rows, acc, isem, gsem, ssem):
    c = lax.axis_index("c")
    s = lax.axis_index("s")

    _zero_vmem(rows, 208, F)
    for k in range(3):
        pltpu.sync_copy(rows.at[pl.ds(0, 208)],
                        acc.at[pl.ds(s * RPP + k * 208, 208)])

    @pl.when(s == NS - 1)
    def _():
        pltpu.sync_copy(rows.at[pl.ds(0, 16)], acc.at[pl.ds(N - 16, 16)])

    if feat_split:
        wpt = NWALL // NS
        base = s * wpt * W
        nw = jnp.where(s == NS - 1, wpt + 4, wpt)
        vsrc = v_hbm.at[c]
    else:
        wpt = NWALL // (NC * NS)
        wid = c * NS + s
        base = wid * wpt * W
        nw = jnp.where(wid == NC * NS - 1, wpt + 4, wpt)
        vsrc = v_hbm

    soff = src_row * NNZ + base
    doff = dst_row * NNZ + base

    def _idesc(g):
        islot = lax.rem(g, 4)
        src_i = idx_hbm.at[pl.ds(soff + g * W, W)]
        dst_i = idx_hbm.at[pl.ds(doff + g * W, W)]
        return (src_i, ib.at[0, islot], dst_i, ib.at[1, islot], islot)

    def _issue_i(g):
        src_i, d0, dst_i, d1, islot = _idesc(g)
        pltpu.async_copy(src_i, d0, isem.at[islot])
        pltpu.async_copy(dst_i, d1, isem.at[islot])

    def _wait_i(g):
        src_i, d0, dst_i, d1, islot = _idesc(g)
        pltpu.make_async_copy(src_i, d0, isem.at[islot]).wait()
        pltpu.make_async_copy(dst_i, d1, isem.at[islot]).wait()

    def _gdesc(g):
        slot = lax.rem(g, 3)
        roff = pl.multiple_of(slot * W, W)
        return (vsrc.at[ib.at[0, lax.rem(g, 4)]], rows.at[pl.ds(roff, W)],
                slot)

    def _issue_g(g):
        src, dst, gslot = _gdesc(g)
        pltpu.async_copy(src, dst, gsem.at[gslot])

    def _wait_g(g):
        src, dst, gslot = _gdesc(g)
        pltpu.make_async_copy(src, dst, gsem.at[gslot]).wait()

    def _sdesc(g):
        slot = lax.rem(g, 3)
        roff = pl.multiple_of(slot * W, W)
        return (rows.at[pl.ds(roff, W)], acc.at[ib.at[1, lax.rem(g, 4)]])

    def _issue_s(g):
        src, dst = _sdesc(g)
        pltpu.async_copy(src, dst, ssem, add=True)

    def _wait_s(g):
        src, dst = _sdesc(g)
        pltpu.make_async_copy(src, dst, ssem).wait()

    plsc.subcore_barrier()

    _issue_i(jnp.int32(0))
    _issue_i(jnp.int32(1))
    _issue_i(jnp.int32(2))
    _wait_i(jnp.int32(0))
    _issue_g(jnp.int32(0))
    _wait_i(jnp.int32(1))
    _issue_g(jnp.int32(1))

    def _win(g, _):
        @pl.when(g >= 1)
        def _():
            _wait_s(g - 1)

        @pl.when(g + 3 < nw)
        def _():
            _issue_i(g + 3)

        @pl.when(g + 2 < nw)
        def _():
            _wait_i(g + 2)
            _issue_g(g + 2)

        _wait_g(g)
        _issue_s(g)
        return 0
    lax.fori_loop(0, nw, _win, 0)
    _wait_s(nw - 1)

    plsc.subcore_barrier()

    @pl.when(s < NS - 1)
    def _():
        pltpu.sync_copy(acc.at[pl.ds(s * RPP, RPP)],
                        out_hbm.at[c, pl.ds(s * RPP, RPP)])

    @pl.when(s == NS - 1)
    def _():
        pltpu.sync_copy(acc.at[pl.ds((NS - 1) * RPP, 640)],
                        out_hbm.at[c, pl.ds((NS - 1) * RPP, 640)])


@functools.cache
def _make_prop(src_row, dst_row, feat_split, tag=0):
    return pl.kernel(
        functools.partial(_prop_body, src_row, dst_row, feat_split),
        out_type=jax.ShapeDtypeStruct((NC, N, F), jnp.float32),
        mesh=_MESH,
        scratch_types=[
            pltpu.VMEM((2, 4, W), jnp.int32),
            pltpu.VMEM((3 * W, F), jnp.float32),
            pltpu.VMEM_SHARED((N, F), jnp.float32),
            pltpu.SemaphoreType.DMA((4,)),
            pltpu.SemaphoreType.DMA((3,)),
            pltpu.SemaphoreType.DMA,
        ],
        name=f"prop_{src_row}{dst_row}{int(feat_split)}_{tag}",
    )


_MB = 2000


def _recip(cnt):
    return jnp.where(cnt > 0, 1.0 / cnt, 0.0)


def _scale_sum_body(p_ref, cnt_ref, o_ref):
    o_ref[...] = (p_ref[0] + p_ref[1]) * _recip(cnt_ref[...])


def _scale_sum(p, cnt):
    return pl.pallas_call(
        _scale_sum_body,
        grid=(N // _MB,),
        in_specs=[
            pl.BlockSpec((NC, _MB, F), lambda m: (0, m, 0)),
            pl.BlockSpec((_MB, 1), lambda m: (m, 0)),
        ],
        out_specs=pl.BlockSpec((_MB, F), lambda m: (m, 0)),
        out_shape=jax.ShapeDtypeStruct((N, F), jnp.float32),
    )(p, cnt)


def _scale_body(e_ref, cnt_ref, o_ref):
    o_ref[...] = e_ref[...] * _recip(cnt_ref[...])


def _scale(e, cnt):
    return pl.pallas_call(
        _scale_body,
        grid=(NC, N // _MB),
        in_specs=[
            pl.BlockSpec((1, _MB, F), lambda h, m: (h, m, 0)),
            pl.BlockSpec((_MB, 1), lambda h, m: (m, 0)),
        ],
        out_specs=pl.BlockSpec((1, _MB, F), lambda h, m: (h, m, 0)),
        out_shape=jax.ShapeDtypeStruct((NC, N, F), jnp.float32),
    )(e, cnt)


def _leaky(y):
    return jnp.where(y > 0, y, 0.01 * y)


def _mm1_body(t_ref, cnt_ref, w_ref, b_ref, o_ref):
    x = (t_ref[0] + t_ref[1]) * _recip(cnt_ref[...])
    y = lax.dot_general(x, w_ref[0], (((1,), (1,)), ((), ())),
                        precision=lax.Precision.HIGHEST)
    o_ref[...] = _leaky(y + b_ref[0])[None]


def _mm1(t, cnt, w_split, b_split):
    return pl.pallas_call(
        _mm1_body,
        grid=(NC, N // _MB),
        in_specs=[
            pl.BlockSpec((NC, _MB, F), lambda h, m: (0, m, 0)),
            pl.BlockSpec((_MB, 1), lambda h, m: (m, 0)),
            pl.BlockSpec((1, F, F), lambda h, m: (h, 0, 0)),
            pl.BlockSpec((1, 1, F), lambda h, m: (h, 0, 0)),
        ],
        out_specs=pl.BlockSpec((1, _MB, F), lambda h, m: (h, m, 0)),
        out_shape=jax.ShapeDtypeStruct((NC, N, F), jnp.float32),
    )(t, cnt, w_split, b_split)


def _mm23_body(t_ref, cnt_ref, w2_ref, b2_ref, w3_ref, o_ref):
    dinv = _recip(cnt_ref[...])
    y = lax.dot_general(t_ref[0] * dinv, w2_ref[0], (((1,), (1,)), ((), ())),
                        precision=lax.Precision.HIGHEST)
    y += lax.dot_general(t_ref[1] * dinv, w2_ref[1], (((1,), (1,)), ((), ())),
                         precision=lax.Precision.HIGHEST)
    h2 = _leaky(y + b2_ref[...])
    o_ref[...] = lax.dot_general(h2, w3_ref[...], (((1,), (1,)), ((), ())),
                                 precision=lax.Precision.HIGHEST)


def _mm23(t, cnt, w2_split, b2, w3):
    return pl.pallas_call(
        _mm23_body,
        grid=(N // _MB,),
        in_specs=[
            pl.BlockSpec((NC, _MB, F), lambda m: (0, m, 0)),
            pl.BlockSpec((_MB, 1), lambda m: (m, 0)),
            pl.BlockSpec((NC, 2 * F, F), lambda m: (0, 0, 0)),
            pl.BlockSpec((1, 2 * F), lambda m: (0, 0)),
            pl.BlockSpec((F, 2 * F), lambda m: (0, 0)),
        ],
        out_specs=pl.BlockSpec((_MB, F), lambda m: (m, 0)),
        out_shape=jax.ShapeDtypeStruct((N, F), jnp.float32),
    )(t, cnt, w2_split, b2, w3)


def _pool_body(t_ref, cnt_ref, b3_ref, wa_ref, o_ref):
    h3 = (t_ref[0] + t_ref[1]) * _recip(cnt_ref[...]) + b3_ref[...]
    s = lax.dot_general(h3, wa_ref[...], (((1,), (1,)), ((), ())),
                        precision=lax.Precision.HIGHEST)
    w = jnp.exp(s - jnp.max(s))
    o_ref[...] = jnp.sum(w * h3, axis=0, keepdims=True) / jnp.sum(w)


def _pool(t, cnt, b3, wa):
    return pl.pallas_call(
        _pool_body,
        grid=(1,),
        in_specs=[
            pl.BlockSpec((NC, N, F), lambda i: (0, 0, 0)),
            pl.BlockSpec((N, 1), lambda i: (0, 0)),
            pl.BlockSpec((1, F), lambda i: (0, 0)),
            pl.BlockSpec((1, F), lambda i: (0, 0)),
        ],
        out_specs=pl.BlockSpec((1, F), lambda i: (0, 0)),
        out_shape=jax.ShapeDtypeStruct((1, F), jnp.float32),
    )(t, cnt, b3, wa)


def kernel(x, hyper_edge_index, W1, b1, W2, b2, W3, b3, Wa, ba):
    del ba
    idx = hyper_edge_index.astype(jnp.int32).reshape(2 * NNZ)

    counts = _counts(idx)
    cnt_n = counts[0, :N, None]
    cnt_e = counts[1, :N, None]

    w1s = W1.reshape(NC, F, F)
    w2s = jnp.stack([W2[:, :F], W2[:, F:]])
    b1s = b1.reshape(NC, 1, F)

    prop_ne_n = _make_prop(0, 1, False)
    prop_en_n = _make_prop(1, 0, False)
    prop_ne_f = _make_prop(0, 1, True)
    prop_en_f = _make_prop(1, 0, True)

    p1 = prop_ne_n(x, idx)
    e1 = _scale_sum(p1, cnt_e)
    t1 = prop_en_n(e1, idx)
    h1 = _mm1(t1, cnt_n, w1s, b1s)

    e2 = _scale(prop_ne_f(h1, idx), cnt_e)
    t2 = prop_en_f(e2, idx)
    g3 = _mm23(t2, cnt_n, w2s, b2[None], W3)

    p3 = _make_prop(0, 1, False, tag=1)(g3, idx)
    e3 = _scale_sum(p3, cnt_e)
    t3 = _make_prop(1, 0, False, tag=1)(e3, idx)

    out = _pool(t3, cnt_n, b3[None], Wa)
    return out[0]
